# Initial kernel scaffold; baseline (speedup 1.0000x reference)
import jax
import jax.numpy as jnp
from jax.experimental import pallas as pl


def kernel(x, edge_index, W1, b1, W2, b2, Wfc, bfc):
    raise NotImplementedError("write your pallas kernel here")


# speedup vs baseline: 19.1996x; 19.1996x over previous
import functools

import jax
import jax.numpy as jnp
from jax import lax
from jax.experimental import pallas as pl
from jax.experimental.pallas import tpu as pltpu
from jax.experimental.pallas import tpu_sc as plsc

N = 10000
E = 320000
D = 128
H = 128
O = 2

NC = 2
NS = 16
NW = NC * NS
EPW = E // NW
K = 80
C = EPW // K
NPAD = 10240
DEG_SLAB = NPAD // NS
ROW_SLAB = NPAD // NS

_mesh = plsc.VectorSubcoreMesh(core_axis_name="c", subcore_axis_name="s")


@functools.partial(
    pl.kernel,
    out_type=jax.ShapeDtypeStruct((NC, NPAD), jnp.float32),
    mesh=_mesh,
    scratch_types=[
        pltpu.VMEM((C, K), jnp.int32),
        pltpu.VMEM((K,), jnp.float32),
        pltpu.VMEM((DEG_SLAB,), jnp.float32),
        pltpu.VMEM_SHARED((NPAD,), jnp.float32),
    ],
)
def _deg_kernel(dst_hbm, out_hbm, idx_v, ones_v, zb_v, acc_sh):
    cid = lax.axis_index("c")
    sid = lax.axis_index("s")
    wid = cid * NS + sid

    def fill_ones(i, carry):
        ones_v[pl.ds(i * 16, 16)] = jnp.ones((16,), jnp.float32)
        return carry

    lax.fori_loop(0, K // 16, fill_ones, 0)

    def fill_zero(i, carry):
        zb_v[pl.ds(i * 16, 16)] = jnp.zeros((16,), jnp.float32)
        return carry

    lax.fori_loop(0, DEG_SLAB // 16, fill_zero, 0)

    pltpu.sync_copy(zb_v, acc_sh.at[pl.ds(sid * DEG_SLAB, DEG_SLAB)])
    plsc.subcore_barrier()

    pltpu.sync_copy(dst_hbm.at[wid], idx_v)

    def chunk(ci, carry):
        pltpu.sync_copy(ones_v, acc_sh.at[idx_v.at[ci]], add=True)
        return carry

    lax.fori_loop(0, C, chunk, 0)
    plsc.subcore_barrier()

    pltpu.sync_copy(
        acc_sh.at[pl.ds(sid * DEG_SLAB, DEG_SLAB)],
        out_hbm.at[cid, pl.ds(sid * DEG_SLAB, DEG_SLAB)],
    )


@functools.partial(
    pl.kernel,
    out_type=jax.ShapeDtypeStruct((NC, NPAD, H), jnp.float32),
    mesh=_mesh,
    scratch_types=[
        pltpu.VMEM((C, K), jnp.int32),
        pltpu.VMEM((C, K), jnp.int32),
        pltpu.VMEM((K, H), jnp.float32),
        pltpu.VMEM_SHARED((NPAD, H), jnp.float32),
        pltpu.SemaphoreType.DMA,
    ],
)
def _edge_kernel(hs_hbm, src_hbm, dst_hbm, zeros_hbm, out_hbm,
                 src_v, dst_v, rows_v, acc_sh, sem):
    cid = lax.axis_index("c")
    sid = lax.axis_index("s")
    wid = cid * NS + sid

    pltpu.sync_copy(
        zeros_hbm.at[pl.ds(sid * ROW_SLAB, ROW_SLAB)],
        acc_sh.at[pl.ds(sid * ROW_SLAB, ROW_SLAB)],
    )
    plsc.subcore_barrier()

    pltpu.sync_copy(src_hbm.at[wid], src_v)
    pltpu.sync_copy(dst_hbm.at[wid], dst_v)

    def chunk(ci, carry):
        pltpu.async_copy(hs_hbm.at[src_v.at[ci]], rows_v, sem).wait()
        pltpu.sync_copy(rows_v, acc_sh.at[dst_v.at[ci]], add=True)
        return carry

    lax.fori_loop(0, C, chunk, 0)
    plsc.subcore_barrier()

    pltpu.sync_copy(
        acc_sh.at[pl.ds(sid * ROW_SLAB, ROW_SLAB)],
        out_hbm.at[cid, pl.ds(sid * ROW_SLAB, ROW_SLAB)],
    )


BN = 1000
GRID = N // BN


def _prep_body(degp_ref, x_ref, w_ref, hs_ref, dinv_ref):
    deg = degp_ref[0] + degp_ref[1] + 1.0
    dinv = lax.rsqrt(deg)
    h = jnp.dot(x_ref[...], w_ref[...], preferred_element_type=jnp.float32)
    hs_ref[...] = h * dinv
    dinv_ref[...] = dinv


def _prep(degp, x, w1):
    return pl.pallas_call(
        _prep_body,
        grid=(GRID,),
        in_specs=[
            pl.BlockSpec((NC, BN, 1), lambda i: (0, i, 0)),
            pl.BlockSpec((BN, D), lambda i: (i, 0)),
            pl.BlockSpec((D, H), lambda i: (0, 0)),
        ],
        out_specs=[
            pl.BlockSpec((BN, H), lambda i: (i, 0)),
            pl.BlockSpec((BN, 1), lambda i: (i, 0)),
        ],
        out_shape=[
            jax.ShapeDtypeStruct((N, H), jnp.float32),
            jax.ShapeDtypeStruct((N, 1), jnp.float32),
        ],
    )(degp, x, w1)


def _mid_body(accp_ref, hs_ref, dinv_ref, b_ref, w_ref, out_ref):
    dinv = dinv_ref[...]
    h = dinv * (accp_ref[0] + accp_ref[1] + hs_ref[...]) + b_ref[...]
    h = jnp.maximum(h, 0.0)
    out_ref[...] = (
        jnp.dot(h, w_ref[...], preferred_element_type=jnp.float32) * dinv
    )


def _mid(accp, hs, dinv, b1, w2):
    return pl.pallas_call(
        _mid_body,
        grid=(GRID,),
        in_specs=[
            pl.BlockSpec((NC, BN, H), lambda i: (0, i, 0)),
            pl.BlockSpec((BN, H), lambda i: (i, 0)),
            pl.BlockSpec((BN, 1), lambda i: (i, 0)),
            pl.BlockSpec((1, H), lambda i: (0, 0)),
            pl.BlockSpec((H, H), lambda i: (0, 0)),
        ],
        out_specs=pl.BlockSpec((BN, H), lambda i: (i, 0)),
        out_shape=jax.ShapeDtypeStruct((N, H), jnp.float32),
    )(accp, hs, dinv, b1, w2)


def _final_body(accp_ref, hs_ref, dinv_ref, b_ref, wfc_ref, bfc_ref,
                out_ref, gsum_ref):
    i = pl.program_id(0)
    h = dinv_ref[...] * (accp_ref[0] + accp_ref[1] + hs_ref[...]) + b_ref[...]
    h = jnp.maximum(h, 0.0)
    part = jnp.sum(h, axis=0, keepdims=True)

    @pl.when(i == 0)
    def _():
        gsum_ref[...] = part

    @pl.when(i > 0)
    def _():
        gsum_ref[...] = gsum_ref[...] + part

    @pl.when(i == pl.num_programs(0) - 1)
    def _():
        g = gsum_ref[...] * (1.0 / N)
        logits = (
            jnp.dot(g, wfc_ref[...], preferred_element_type=jnp.float32)
            + bfc_ref[...]
        )
        m = jnp.max(logits, axis=1, keepdims=True)
        z = jnp.log(jnp.sum(jnp.exp(logits - m), axis=1, keepdims=True))
        out_ref[...] = logits - m - z


def _final(accp, hs, dinv, b2, wfc, bfc):
    return pl.pallas_call(
        _final_body,
        grid=(GRID,),
        in_specs=[
            pl.BlockSpec((NC, BN, H), lambda i: (0, i, 0)),
            pl.BlockSpec((BN, H), lambda i: (i, 0)),
            pl.BlockSpec((BN, 1), lambda i: (i, 0)),
            pl.BlockSpec((1, H), lambda i: (0, 0)),
            pl.BlockSpec((H, O), lambda i: (0, 0)),
            pl.BlockSpec((1, O), lambda i: (0, 0)),
        ],
        out_specs=pl.BlockSpec((1, O), lambda i: (0, 0)),
        out_shape=jax.ShapeDtypeStruct((1, O), jnp.float32),
        scratch_shapes=[pltpu.VMEM((1, H), jnp.float32)],
    )(accp, hs, dinv, b2, wfc, bfc)


def kernel(x, edge_index, W1, b1, W2, b2, Wfc, bfc):
    src3 = edge_index[0].reshape(NW, C, K)
    dst3 = edge_index[1].reshape(NW, C, K)
    zeros = jnp.zeros((NPAD, H), jnp.float32)

    degp = _deg_kernel(dst3)[:, :N].reshape(NC, N, 1)

    hs1, dinv = _prep(degp, x, W1)
    acc1 = _edge_kernel(hs1, src3, dst3, zeros)[:, :N]
    hs2 = _mid(acc1, hs1, dinv, b1.reshape(1, H), W2)
    acc2 = _edge_kernel(hs2, src3, dst3, zeros)[:, :N]
    return _final(acc2, hs2, dinv, b2.reshape(1, H), Wfc, bfc.reshape(1, O))


# speedup vs baseline: 27.4724x; 1.4309x over previous
import functools

import jax
import jax.numpy as jnp
from jax import lax
from jax.experimental import pallas as pl
from jax.experimental.pallas import tpu as pltpu
from jax.experimental.pallas import tpu_sc as plsc

N = 10000
E = 320000
D = 128
H = 128
O = 2

NC = 2
NS = 16
NW = NC * NS
EPW = E // NW
K = 80
C = EPW // K
KE = 125
CE = EPW // KE
NPAD = 10240
DEG_SLAB = NPAD // NS
ROW_SLAB = NPAD // NS

_mesh = plsc.VectorSubcoreMesh(core_axis_name="c", subcore_axis_name="s")


@functools.partial(
    pl.kernel,
    out_type=jax.ShapeDtypeStruct((NC, NPAD), jnp.float32),
    mesh=_mesh,
    scratch_types=[
        pltpu.VMEM((C, K), jnp.int32),
        pltpu.VMEM((K,), jnp.float32),
        pltpu.VMEM((DEG_SLAB,), jnp.float32),
        pltpu.VMEM_SHARED((NPAD,), jnp.float32),
    ],
)
def _deg_kernel(dst_hbm, out_hbm, idx_v, ones_v, zb_v, acc_sh):
    cid = lax.axis_index("c")
    sid = lax.axis_index("s")
    wid = cid * NS + sid

    def fill_ones(i, carry):
        ones_v[pl.ds(i * 16, 16)] = jnp.ones((16,), jnp.float32)
        return carry

    lax.fori_loop(0, K // 16, fill_ones, 0)

    def fill_zero(i, carry):
        zb_v[pl.ds(i * 16, 16)] = jnp.zeros((16,), jnp.float32)
        return carry

    lax.fori_loop(0, DEG_SLAB // 16, fill_zero, 0)

    pltpu.sync_copy(zb_v, acc_sh.at[pl.ds(sid * DEG_SLAB, DEG_SLAB)])
    plsc.subcore_barrier()

    pltpu.sync_copy(dst_hbm.at[wid], idx_v)

    def chunk(ci, carry):
        pltpu.sync_copy(ones_v, acc_sh.at[idx_v.at[ci]], add=True)
        return carry

    lax.fori_loop(0, C, chunk, 0)
    plsc.subcore_barrier()

    pltpu.sync_copy(
        acc_sh.at[pl.ds(sid * DEG_SLAB, DEG_SLAB)],
        out_hbm.at[cid, pl.ds(sid * DEG_SLAB, DEG_SLAB)],
    )


@functools.partial(
    pl.kernel,
    out_type=jax.ShapeDtypeStruct((NC, NPAD, H), jnp.float32),
    mesh=_mesh,
    scratch_types=[
        pltpu.VMEM((2, KE), jnp.int32),
        pltpu.VMEM((2, KE), jnp.int32),
        pltpu.VMEM((KE, H), jnp.float32),
        pltpu.VMEM((KE, H), jnp.float32),
        pltpu.VMEM_SHARED((NPAD, H), jnp.float32),
        pltpu.SemaphoreType.DMA,
        pltpu.SemaphoreType.DMA,
        pltpu.SemaphoreType.DMA,
        pltpu.SemaphoreType.DMA,
    ],
)
def _edge_kernel(hs_hbm, idx_hbm, zeros_hbm, out_hbm,
                 idx_a, idx_b, rows_a, rows_b, acc_sh,
                 sem_ia, sem_ib, sem_ga, sem_gb):
    cid = lax.axis_index("c")
    sid = lax.axis_index("s")
    wid = cid * NS + sid

    pltpu.sync_copy(
        zeros_hbm.at[pl.ds(sid * ROW_SLAB, ROW_SLAB)],
        acc_sh.at[pl.ds(sid * ROW_SLAB, ROW_SLAB)],
    )
    plsc.subcore_barrier()

    pltpu.async_copy(idx_hbm.at[wid, 0], idx_a, sem_ia)
    pltpu.make_async_copy(idx_hbm.at[wid, 0], idx_a, sem_ia).wait()
    pltpu.async_copy(hs_hbm.at[idx_a.at[0]], rows_a, sem_ga)
    pltpu.async_copy(idx_hbm.at[wid, 1], idx_b, sem_ib)

    def pair(p, carry):
        ca = 2 * p
        pltpu.make_async_copy(hs_hbm.at[idx_a.at[0]], rows_a, sem_ga).wait()
        pltpu.make_async_copy(idx_hbm.at[wid, ca + 1], idx_b, sem_ib).wait()
        pltpu.async_copy(hs_hbm.at[idx_b.at[0]], rows_b, sem_gb)
        pltpu.sync_copy(rows_a, acc_sh.at[idx_a.at[1]], add=True)

        @pl.when(ca + 2 < CE)
        def _():
            pltpu.async_copy(idx_hbm.at[wid, ca + 2], idx_a, sem_ia)

        pltpu.make_async_copy(hs_hbm.at[idx_b.at[0]], rows_b, sem_gb).wait()

        @pl.when(ca + 2 < CE)
        def _():
            pltpu.make_async_copy(
                idx_hbm.at[wid, ca + 2], idx_a, sem_ia).wait()
            pltpu.async_copy(hs_hbm.at[idx_a.at[0]], rows_a, sem_ga)

        pltpu.sync_copy(rows_b, acc_sh.at[idx_b.at[1]], add=True)

        @pl.when(ca + 3 < CE)
        def _():
            pltpu.async_copy(idx_hbm.at[wid, ca + 3], idx_b, sem_ib)

        return carry

    lax.fori_loop(0, CE // 2, pair, 0)
    plsc.subcore_barrier()

    pltpu.sync_copy(
        acc_sh.at[pl.ds(sid * ROW_SLAB, ROW_SLAB)],
        out_hbm.at[cid, pl.ds(sid * ROW_SLAB, ROW_SLAB)],
    )


BN = 1000
GRID = N // BN


def _prep_body(degp_ref, x_ref, w_ref, hs_ref, dinv_ref):
    deg = degp_ref[0] + degp_ref[1] + 1.0
    dinv = lax.rsqrt(deg)
    h = jnp.dot(x_ref[...], w_ref[...], preferred_element_type=jnp.float32)
    hs_ref[...] = h * dinv
    dinv_ref[...] = dinv


def _prep(degp, x, w1):
    return pl.pallas_call(
        _prep_body,
        grid=(GRID,),
        in_specs=[
            pl.BlockSpec((NC, BN, 1), lambda i: (0, i, 0)),
            pl.BlockSpec((BN, D), lambda i: (i, 0)),
            pl.BlockSpec((D, H), lambda i: (0, 0)),
        ],
        out_specs=[
            pl.BlockSpec((BN, H), lambda i: (i, 0)),
            pl.BlockSpec((BN, 1), lambda i: (i, 0)),
        ],
        out_shape=[
            jax.ShapeDtypeStruct((N, H), jnp.float32),
            jax.ShapeDtypeStruct((N, 1), jnp.float32),
        ],
    )(degp, x, w1)


def _mid_body(accp_ref, hs_ref, dinv_ref, b_ref, w_ref, out_ref):
    dinv = dinv_ref[...]
    h = dinv * (accp_ref[0] + accp_ref[1] + hs_ref[...]) + b_ref[...]
    h = jnp.maximum(h, 0.0)
    out_ref[...] = (
        jnp.dot(h, w_ref[...], preferred_element_type=jnp.float32) * dinv
    )


def _mid(accp, hs, dinv, b1, w2):
    return pl.pallas_call(
        _mid_body,
        grid=(GRID,),
        in_specs=[
            pl.BlockSpec((NC, BN, H), lambda i: (0, i, 0)),
            pl.BlockSpec((BN, H), lambda i: (i, 0)),
            pl.BlockSpec((BN, 1), lambda i: (i, 0)),
            pl.BlockSpec((1, H), lambda i: (0, 0)),
            pl.BlockSpec((H, H), lambda i: (0, 0)),
        ],
        out_specs=pl.BlockSpec((BN, H), lambda i: (i, 0)),
        out_shape=jax.ShapeDtypeStruct((N, H), jnp.float32),
    )(accp, hs, dinv, b1, w2)


def _final_body(accp_ref, hs_ref, dinv_ref, b_ref, wfc_ref, bfc_ref,
                out_ref, gsum_ref):
    i = pl.program_id(0)
    h = dinv_ref[...] * (accp_ref[0] + accp_ref[1] + hs_ref[...]) + b_ref[...]
    h = jnp.maximum(h, 0.0)
    part = jnp.sum(h, axis=0, keepdims=True)

    @pl.when(i == 0)
    def _():
        gsum_ref[...] = part

    @pl.when(i > 0)
    def _():
        gsum_ref[...] = gsum_ref[...] + part

    @pl.when(i == pl.num_programs(0) - 1)
    def _():
        g = gsum_ref[...] * (1.0 / N)
        logits = (
            jnp.dot(g, wfc_ref[...], preferred_element_type=jnp.float32)
            + bfc_ref[...]
        )
        m = jnp.max(logits, axis=1, keepdims=True)
        z = jnp.log(jnp.sum(jnp.exp(logits - m), axis=1, keepdims=True))
        out_ref[...] = logits - m - z


def _final(accp, hs, dinv, b2, wfc, bfc):
    return pl.pallas_call(
        _final_body,
        grid=(GRID,),
        in_specs=[
            pl.BlockSpec((NC, BN, H), lambda i: (0, i, 0)),
            pl.BlockSpec((BN, H), lambda i: (i, 0)),
            pl.BlockSpec((BN, 1), lambda i: (i, 0)),
            pl.BlockSpec((1, H), lambda i: (0, 0)),
            pl.BlockSpec((H, O), lambda i: (0, 0)),
            pl.BlockSpec((1, O), lambda i: (0, 0)),
        ],
        out_specs=pl.BlockSpec((1, O), lambda i: (0, 0)),
        out_shape=jax.ShapeDtypeStruct((1, O), jnp.float32),
        scratch_shapes=[pltpu.VMEM((1, H), jnp.float32)],
    )(accp, hs, dinv, b2, wfc, bfc)


def kernel(x, edge_index, W1, b1, W2, b2, Wfc, bfc):
    idx4 = jnp.stack(
        [edge_index[0].reshape(NW, CE, KE),
         edge_index[1].reshape(NW, CE, KE)], axis=2)
    dst3_deg = edge_index[1].reshape(NW, C, K)
    zeros = jnp.zeros((NPAD, H), jnp.float32)

    degp = _deg_kernel(dst3_deg)[:, :N].reshape(NC, N, 1)

    hs1, dinv = _prep(degp, x, W1)
    acc1 = _edge_kernel(hs1, idx4, zeros)[:, :N]
    hs2 = _mid(acc1, hs1, dinv, b1.reshape(1, H), W2)
    acc2 = _edge_kernel(hs2, idx4, zeros)[:, :N]
    return _final(acc2, hs2, dinv, b2.reshape(1, H), Wfc, bfc.reshape(1, O))
